# baseline (speedup 1.0000x reference)
import functools

import jax
import jax.numpy as jnp
from jax import lax
from jax.experimental import pallas as pl
from jax.experimental.pallas import tpu as pltpu
from jax.experimental.pallas import tpu_sc as plsc

NC = 2
NS = 16
LANES = 16
CH = 128


def _spmm_sc(feat, edata, wdata, zeros_tile, p0):
    d = feat.shape[1]
    rpt = zeros_tile.shape[0]
    n_pad = rpt * NS
    s_pairs = edata.shape[0] // NS
    p1 = s_pairs - p0
    assert p0 % 2 == 0 and p1 % 2 == 0 and p0 >= 2 and p1 >= 2

    mesh = plsc.VectorSubcoreMesh(
        core_axis_name="c", subcore_axis_name="s", num_cores=NC,
        num_subcores=NS)

    @functools.partial(
        pl.kernel,
        mesh=mesh,
        out_type=jax.ShapeDtypeStruct((NC, n_pad, d), jnp.float32),
        scratch_types=[
            pltpu.VMEM((2, 2, CH), jnp.int32),
            pltpu.VMEM((2, 2, CH), jnp.int32),
            pltpu.VMEM((2, CH), jnp.float32),
            pltpu.VMEM((2, CH), jnp.float32),
            pltpu.VMEM((CH, d), jnp.float32),
            pltpu.VMEM((CH, d), jnp.float32),
            pltpu.VMEM_SHARED((n_pad, d), jnp.float32),
            pltpu.SemaphoreType.DMA,
            pltpu.SemaphoreType.DMA,
            pltpu.SemaphoreType.DMA,
            pltpu.SemaphoreType.DMA,
        ],
        compiler_params=pltpu.CompilerParams(use_tc_tiling_on_sc=False),
    )
    def spmm_kernel(feat_hbm, ed_hbm, wd_hbm, zero_hbm, out_hbm,
                    eb0, eb1, wb0, wb1, rowsA, rowsB, acc,
                    sE0, sE1, sGA, sGB):
        cid = lax.axis_index("c")
        sid = lax.axis_index("s")
        p_loc = jnp.where(cid == 0, p0, p1)
        pbase = jnp.where(cid == 0, sid * p0, NS * p0 + sid * p1)
        nsteps = p_loc // 2

        pltpu.sync_copy(zero_hbm, acc.at[pl.ds(sid * rpt, rpt)])
        plsc.subcore_barrier()

        def mul(rows, wb, j):
            @plsc.parallel_loop(0, CH // LANES, unroll=2)
            def mul_body(grp):
                wgrp = wb[j, pl.ds(grp * LANES, LANES)]
                for t in range(LANES):
                    w = wgrp[t]
                    e = grp * LANES + t
                    for k in range(d // LANES):
                        sl = pl.ds(k * LANES, LANES)
                        rows[e, sl] = rows[e, sl] * w

        def fire_e(eb, wb, sem, p):
            pltpu.async_copy(ed_hbm.at[pbase + p], eb, sem)
            pltpu.async_copy(wd_hbm.at[pbase + p], wb, sem)

        def wait_e(eb, wb, sem):
            pltpu.make_async_copy(ed_hbm.at[pbase], eb, sem).wait()
            pltpu.make_async_copy(wd_hbm.at[pbase], wb, sem).wait()

        def fire_g(eb, j, rows, sem):
            pltpu.async_copy(feat_hbm.at[eb.at[j, 1]], rows, sem)

        def wait_g(eb, rows, sem):
            pltpu.make_async_copy(feat_hbm.at[eb.at[0, 1]], rows, sem).wait()

        fire_e(eb0, wb0, sE0, 0)
        fire_e(eb1, wb1, sE1, 1)
        wait_e(eb0, wb0, sE0)
        fire_g(eb0, 0, rowsA, sGA)

        def half(eb, wb, s_this, other_eb, other_wb, s_other, refill_p,
                 more):
            fire_g(eb, 1, rowsB, sGB)
            wait_g(eb, rowsA, sGA)
            mul(rowsA, wb, 0)
            pltpu.sync_copy(rowsA, acc.at[eb.at[0, 0]], add=True)

            @pl.when(more)
            def _():
                wait_e(other_eb, other_wb, s_other)
                fire_g(other_eb, 0, rowsA, sGA)

            wait_g(eb, rowsB, sGB)
            mul(rowsB, wb, 1)
            pltpu.sync_copy(rowsB, acc.at[eb.at[1, 0]], add=True)

            @pl.when(refill_p < p_loc)
            def _():
                fire_e(eb, wb, s_this, refill_p)

        def step(s, carry):
            last = s >= nsteps - 1
            half(eb0, wb0, sE0, eb1, wb1, sE1, 2 * s + 2, True)
            half(eb1, wb1, sE1, eb0, wb0, sE0, 2 * s + 3,
                 jnp.logical_not(last))
            return carry

        lax.fori_loop(0, nsteps, step, 0)

        plsc.subcore_barrier()
        pltpu.sync_copy(acc.at[pl.ds(sid * rpt, rpt)],
                        out_hbm.at[cid, pl.ds(sid * rpt, rpt)])

    return spmm_kernel(feat, edata, wdata, zeros_tile)


def _dense_in_body(x_ref, w_ref, h1_ref, p_ref):
    m = jnp.dot(x_ref[...], w_ref[...], preferred_element_type=jnp.float32)
    h1_ref[...] = jnp.maximum(m[:, :64], 0.0)
    p_ref[...] = m[:, 64:]


def _combine_body(p_ref, h2_ref, t3_ref):
    s = p_ref[0] + p_ref[1]
    h2_ref[...] = jnp.maximum(s[:, :64], 0.0)
    t3_ref[...] = s[:, 64:]


def _final_body(h1_ref, h2_ref, q_ref, fcw_ref, fcb_ref, out_ref):
    h3 = jnp.maximum(q_ref[0] + q_ref[1], 0.0)
    a = jnp.concatenate([h1_ref[...], h2_ref[...], h3], axis=1)
    logits = jnp.dot(a, fcw_ref[...], preferred_element_type=jnp.float32)
    logits = logits + fcb_ref[...]
    ncls = 40
    colid = lax.broadcasted_iota(jnp.int32, logits.shape, 1)
    logits = jnp.where(colid < ncls, logits, -jnp.inf)
    m = jnp.max(logits, axis=1, keepdims=True)
    ex = jnp.exp(logits - m)
    s = jnp.sum(ex, axis=1, keepdims=True)
    out_ref[...] = logits - m - jnp.log(s)


def kernel(features, edge_index, edge_weight, W1, W2, W3, fc_w, fc_b):
    n, dfeat = features.shape
    e = edge_index.shape[1]
    d1 = W1.shape[1]
    d23 = W2.shape[1] + W3.shape[1]
    d3 = W3.shape[1]
    ncls = fc_w.shape[1]
    nw = NC * NS

    rpt = -(-n // (NS * 8)) * 8
    n_pad = rpt * NS

    s_pairs = -(-e // (NS * 2 * CH * 2)) * 2
    n_pairs = NS * s_pairs
    e_pad = n_pairs * 2 * CH
    row1 = jnp.pad(edge_index[0], (0, e_pad - e)).reshape(n_pairs, 2, CH)
    col1 = jnp.pad(edge_index[1], (0, e_pad - e)).reshape(n_pairs, 2, CH)
    edata = jnp.stack([row1, col1], axis=2)
    wdata = jnp.pad(edge_weight, (0, e_pad - e)).reshape(n_pairs, 2, CH)
    p0 = 76

    wcat = jnp.concatenate([W1, W2, W3], axis=1)

    blk = 2000
    grid = n // blk
    h1, p = pl.pallas_call(
        _dense_in_body,
        grid=(grid,),
        in_specs=[
            pl.BlockSpec((blk, dfeat), lambda i: (i, 0)),
            pl.BlockSpec((dfeat, d1 + d23), lambda i: (0, 0)),
        ],
        out_specs=[
            pl.BlockSpec((blk, d1), lambda i: (i, 0)),
            pl.BlockSpec((blk, d23), lambda i: (i, 0)),
        ],
        out_shape=[
            jax.ShapeDtypeStruct((n, d1), jnp.float32),
            jax.ShapeDtypeStruct((n, d23), jnp.float32),
        ],
    )(features, wcat)

    zeros128 = jnp.zeros((rpt, d23), jnp.float32)
    part1 = _spmm_sc(p, edata, wdata, zeros128, p0)

    h2, t3 = pl.pallas_call(
        _combine_body,
        grid=(NS,),
        in_specs=[pl.BlockSpec((NC, rpt, d23), lambda i: (0, i, 0))],
        out_specs=[
            pl.BlockSpec((rpt, d1), lambda i: (i, 0)),
            pl.BlockSpec((rpt, d3), lambda i: (i, 0)),
        ],
        out_shape=[
            jax.ShapeDtypeStruct((n_pad, d1), jnp.float32),
            jax.ShapeDtypeStruct((n_pad, d3), jnp.float32),
        ],
    )(part1)

    zeros64 = jnp.zeros((rpt, d3), jnp.float32)
    part2 = _spmm_sc(t3, edata, wdata, zeros64, p0)

    npad = 128
    fcw_pad = jnp.zeros((fc_w.shape[0], npad), jnp.float32).at[:, :ncls].set(fc_w)
    fcb_pad = jnp.zeros((1, npad), jnp.float32).at[0, :ncls].set(fc_b)
    out_pad = pl.pallas_call(
        _final_body,
        grid=(grid,),
        in_specs=[
            pl.BlockSpec((blk, d1), lambda i: (i, 0)),
            pl.BlockSpec((blk, d1), lambda i: (i, 0)),
            pl.BlockSpec((NC, blk, d3), lambda i: (0, i, 0)),
            pl.BlockSpec((fc_w.shape[0], npad), lambda i: (0, 0)),
            pl.BlockSpec((1, npad), lambda i: (0, 0)),
        ],
        out_specs=pl.BlockSpec((blk, npad), lambda i: (i, 0)),
        out_shape=jax.ShapeDtypeStruct((n, npad), jnp.float32),
    )(h1, h2, part2, fcw_pad, fcb_pad)
    return out_pad[:, :ncls]


# speedup vs baseline: 1.0216x; 1.0216x over previous
import functools

import jax
import jax.numpy as jnp
from jax import lax
from jax.experimental import pallas as pl
from jax.experimental.pallas import tpu as pltpu
from jax.experimental.pallas import tpu_sc as plsc

NC = 2
NS = 16
LANES = 16
CH = 128


def _spmm_sc(feat, edata, wdata, zeros_tile, p0):
    d = feat.shape[1]
    rpt = zeros_tile.shape[0]
    n_pad = rpt * NS
    s_pairs = edata.shape[0] // NS
    p1 = s_pairs - p0
    assert p0 % 2 == 0 and p1 % 2 == 0 and p0 >= 2 and p1 >= 2

    mesh = plsc.VectorSubcoreMesh(
        core_axis_name="c", subcore_axis_name="s", num_cores=NC,
        num_subcores=NS)

    @functools.partial(
        pl.kernel,
        mesh=mesh,
        out_type=jax.ShapeDtypeStruct((NC, n_pad, d), jnp.float32),
        scratch_types=[
            pltpu.VMEM((2, 2, CH), jnp.int32),
            pltpu.VMEM((2, 2, CH), jnp.int32),
            pltpu.VMEM((2, CH), jnp.float32),
            pltpu.VMEM((2, CH), jnp.float32),
            pltpu.VMEM((CH, d), jnp.float32),
            pltpu.VMEM((CH, d), jnp.float32),
            pltpu.VMEM_SHARED((n_pad, d), jnp.float32),
            pltpu.SemaphoreType.DMA,
            pltpu.SemaphoreType.DMA,
            pltpu.SemaphoreType.DMA,
            pltpu.SemaphoreType.DMA,
        ],
        compiler_params=pltpu.CompilerParams(use_tc_tiling_on_sc=False),
    )
    def spmm_kernel(feat_hbm, ed_hbm, wd_hbm, zero_hbm, out_hbm,
                    eb0, eb1, wb0, wb1, rowsA, rowsB, acc,
                    sE0, sE1, sGA, sGB):
        cid = lax.axis_index("c")
        sid = lax.axis_index("s")
        p_loc = jnp.where(cid == 0, p0, p1)
        pbase = jnp.where(cid == 0, sid * p0, NS * p0 + sid * p1)
        nsteps = p_loc // 2

        pltpu.sync_copy(zero_hbm, acc.at[pl.ds(sid * rpt, rpt)])
        plsc.subcore_barrier()

        def mul(rows, wb, j):
            @plsc.parallel_loop(0, CH // LANES, unroll=2)
            def mul_body(grp):
                wgrp = wb[j, pl.ds(grp * LANES, LANES)]
                for t in range(LANES):
                    w = wgrp[t]
                    e = grp * LANES + t
                    for k in range(d // LANES):
                        sl = pl.ds(k * LANES, LANES)
                        rows[e, sl] = rows[e, sl] * w

        def fire_e(eb, wb, sem, p):
            pltpu.async_copy(ed_hbm.at[pbase + p], eb, sem)
            pltpu.async_copy(wd_hbm.at[pbase + p], wb, sem)

        def wait_e(eb, wb, sem):
            pltpu.make_async_copy(ed_hbm.at[pbase], eb, sem).wait()
            pltpu.make_async_copy(wd_hbm.at[pbase], wb, sem).wait()

        def fire_g(eb, j, rows, sem):
            pltpu.async_copy(feat_hbm.at[eb.at[j, 1]], rows, sem)

        def wait_g(eb, rows, sem):
            pltpu.make_async_copy(feat_hbm.at[eb.at[0, 1]], rows, sem).wait()

        fire_e(eb0, wb0, sE0, 0)
        fire_e(eb1, wb1, sE1, 1)
        wait_e(eb0, wb0, sE0)
        fire_g(eb0, 0, rowsA, sGA)

        def half(eb, wb, s_this, other_eb, other_wb, s_other, refill_p,
                 more):
            fire_g(eb, 1, rowsB, sGB)
            wait_g(eb, rowsA, sGA)
            mul(rowsA, wb, 0)
            pltpu.sync_copy(rowsA, acc.at[eb.at[0, 0]], add=True)

            @pl.when(more)
            def _():
                wait_e(other_eb, other_wb, s_other)
                fire_g(other_eb, 0, rowsA, sGA)

            wait_g(eb, rowsB, sGB)
            mul(rowsB, wb, 1)
            pltpu.sync_copy(rowsB, acc.at[eb.at[1, 0]], add=True)

            @pl.when(refill_p < p_loc)
            def _():
                fire_e(eb, wb, s_this, refill_p)

        def step(s, carry):
            last = s >= nsteps - 1
            half(eb0, wb0, sE0, eb1, wb1, sE1, 2 * s + 2, True)
            half(eb1, wb1, sE1, eb0, wb0, sE0, 2 * s + 3,
                 jnp.logical_not(last))
            return carry

        lax.fori_loop(0, nsteps, step, 0)

        plsc.subcore_barrier()
        pltpu.sync_copy(acc.at[pl.ds(sid * rpt, rpt)],
                        out_hbm.at[cid, pl.ds(sid * rpt, rpt)])

    return spmm_kernel(feat, edata, wdata, zeros_tile)


def _dense_in_body(x_ref, w_ref, h1_ref, p_ref):
    m = jnp.dot(x_ref[...], w_ref[...], preferred_element_type=jnp.float32)
    h1_ref[...] = jnp.maximum(m[:, :64], 0.0)
    p_ref[...] = m[:, 64:]


def _combine_body(p_ref, h2_ref, t3_ref):
    s = p_ref[0] + p_ref[1]
    h2_ref[...] = jnp.maximum(s[:, :64], 0.0)
    t3_ref[...] = s[:, 64:]


def _final_body(h1_ref, h2_ref, q_ref, fcw_ref, fcb_ref, out_ref):
    h3 = jnp.maximum(q_ref[0] + q_ref[1], 0.0)
    a = jnp.concatenate([h1_ref[...], h2_ref[...], h3], axis=1)
    logits = jnp.dot(a, fcw_ref[...], preferred_element_type=jnp.float32)
    logits = logits + fcb_ref[...]
    ncls = 40
    colid = lax.broadcasted_iota(jnp.int32, logits.shape, 1)
    logits = jnp.where(colid < ncls, logits, -jnp.inf)
    m = jnp.max(logits, axis=1, keepdims=True)
    ex = jnp.exp(logits - m)
    s = jnp.sum(ex, axis=1, keepdims=True)
    out_ref[...] = logits - m - jnp.log(s)


def kernel(features, edge_index, edge_weight, W1, W2, W3, fc_w, fc_b):
    n, dfeat = features.shape
    e = edge_index.shape[1]
    d1 = W1.shape[1]
    d23 = W2.shape[1] + W3.shape[1]
    d3 = W3.shape[1]
    ncls = fc_w.shape[1]
    nw = NC * NS

    rpt = -(-n // (NS * 8)) * 8
    n_pad = rpt * NS

    s_pairs = -(-e // (NS * 2 * CH * 2)) * 2
    n_pairs = NS * s_pairs
    e_pad = n_pairs * 2 * CH
    row1 = jnp.pad(edge_index[0], (0, e_pad - e)).reshape(n_pairs, 2, CH)
    col1 = jnp.pad(edge_index[1], (0, e_pad - e)).reshape(n_pairs, 2, CH)
    edata = jnp.stack([row1, col1], axis=2)
    wdata = jnp.pad(edge_weight, (0, e_pad - e)).reshape(n_pairs, 2, CH)
    p0 = 74

    wcat = jnp.concatenate([W1, W2, W3], axis=1)

    blk = 2000
    grid = n // blk
    h1, p = pl.pallas_call(
        _dense_in_body,
        grid=(grid,),
        in_specs=[
            pl.BlockSpec((blk, dfeat), lambda i: (i, 0)),
            pl.BlockSpec((dfeat, d1 + d23), lambda i: (0, 0)),
        ],
        out_specs=[
            pl.BlockSpec((blk, d1), lambda i: (i, 0)),
            pl.BlockSpec((blk, d23), lambda i: (i, 0)),
        ],
        out_shape=[
            jax.ShapeDtypeStruct((n, d1), jnp.float32),
            jax.ShapeDtypeStruct((n, d23), jnp.float32),
        ],
    )(features, wcat)

    zeros128 = jnp.zeros((rpt, d23), jnp.float32)
    part1 = _spmm_sc(p, edata, wdata, zeros128, p0)

    h2, t3 = pl.pallas_call(
        _combine_body,
        grid=(NS,),
        in_specs=[pl.BlockSpec((NC, rpt, d23), lambda i: (0, i, 0))],
        out_specs=[
            pl.BlockSpec((rpt, d1), lambda i: (i, 0)),
            pl.BlockSpec((rpt, d3), lambda i: (i, 0)),
        ],
        out_shape=[
            jax.ShapeDtypeStruct((n_pad, d1), jnp.float32),
            jax.ShapeDtypeStruct((n_pad, d3), jnp.float32),
        ],
    )(part1)

    zeros64 = jnp.zeros((rpt, d3), jnp.float32)
    part2 = _spmm_sc(t3, edata, wdata, zeros64, p0)

    npad = 128
    fcw_pad = jnp.zeros((fc_w.shape[0], npad), jnp.float32).at[:, :ncls].set(fc_w)
    fcb_pad = jnp.zeros((1, npad), jnp.float32).at[0, :ncls].set(fc_b)
    out_pad = pl.pallas_call(
        _final_body,
        grid=(grid,),
        in_specs=[
            pl.BlockSpec((blk, d1), lambda i: (i, 0)),
            pl.BlockSpec((blk, d1), lambda i: (i, 0)),
            pl.BlockSpec((NC, blk, d3), lambda i: (0, i, 0)),
            pl.BlockSpec((fc_w.shape[0], npad), lambda i: (0, 0)),
            pl.BlockSpec((1, npad), lambda i: (0, 0)),
        ],
        out_specs=pl.BlockSpec((blk, npad), lambda i: (i, 0)),
        out_shape=jax.ShapeDtypeStruct((n, npad), jnp.float32),
    )(h1, h2, part2, fcw_pad, fcb_pad)
    return out_pad[:, :ncls]
